# baseline (speedup 1.0000x reference)
import functools
import jax
import jax.numpy as jnp
from jax.experimental import pallas as pl
from jax.experimental.pallas import tpu as pltpu

N = 100000
ROWS = 800


def _mm_body(x_ref, w_ref, o_ref):
    o_ref[...] = jnp.dot(x_ref[...], w_ref[...], preferred_element_type=jnp.float32)


def _matmul(x, w):
    n, k = x.shape
    _, m = w.shape
    grid = n // ROWS
    return pl.pallas_call(
        _mm_body,
        grid=(grid,),
        in_specs=[
            pl.BlockSpec((ROWS, k), lambda i: (i, 0)),
            pl.BlockSpec((k, m), lambda i: (0, 0)),
        ],
        out_specs=pl.BlockSpec((ROWS, m), lambda i: (i, 0)),
        out_shape=jax.ShapeDtypeStruct((n, m), jnp.float32),
    )(x, w)


def _gcn_conv(x, W, b, src, dst, dinv):
    h = _matmul(x, W)
    norm = dinv[src] * dinv[dst]
    msgs = h[src] * norm[:, None]
    out = jnp.zeros_like(h).at[dst].add(msgs)
    out = out + h * (dinv * dinv)[:, None]
    return out + b


@jax.jit
def kernel(x, ei, targets, W1, b1, W2, b2, Wout, bout):
    src, dst = ei[0], ei[1]
    deg = jnp.zeros((N,), dtype=jnp.float32).at[dst].add(1.0) + 1.0
    dinv = jax.lax.rsqrt(deg)
    h = _gcn_conv(x, W1, b1, src, dst, dinv)
    h = jax.nn.relu(h)
    z = _gcn_conv(h, W2, b2, src, dst, dinv)
    s = z[targets[0]]
    d = z[targets[1]]
    return (s * d) @ Wout + bout


# speedup vs baseline: 14.0471x; 14.0471x over previous
import functools
import jax
import jax.numpy as jnp
from jax import lax
from jax.experimental import pallas as pl
from jax.experimental.pallas import tpu as pltpu
from jax.experimental.pallas import tpu_sc as plsc

N = 100000
NP = N + 16
E = 3200000
BLK = 1024
CHUNK = 128
NCHUNK = BLK // CHUNK
E_PAD = 3211264
EP_TILE = E_PAD // 16
NB = EP_TILE // BLK
NS = 100096
RPT = NS // 16
_ZCHUNKS = tuple((k * BLK, BLK) for k in range(6)) + ((6 * BLK, 112),)

ROWS = 800


def _mm_body(x_ref, w_ref, o_ref):
    o_ref[...] = jnp.dot(x_ref[...], w_ref[...], preferred_element_type=jnp.float32)


def _matmul(x, w):
    n, k = x.shape
    _, m = w.shape
    return pl.pallas_call(
        _mm_body,
        grid=(n // ROWS,),
        in_specs=[
            pl.BlockSpec((ROWS, k), lambda i: (i, 0)),
            pl.BlockSpec((k, m), lambda i: (0, 0)),
        ],
        out_specs=pl.BlockSpec((ROWS, m), lambda i: (i, 0)),
        out_shape=jax.ShapeDtypeStruct((n, m), jnp.float32),
    )(x, w)


def _spmm_body(spc, g_hbm, src_hbm, dst_hbm, out_hbm,
               src_v, dst_v, rows_v, acc_sh, gsem):
    c = lax.axis_index("c")
    tid = lax.axis_index("s")
    row0 = tid * RPT

    for sl in range(spc):
        slice_id = c * spc + sl
        off = slice_id * NP

        def zb(k, carry):
            rows_v[k, :] = jnp.zeros((16,), jnp.float32)
            return carry
        lax.fori_loop(0, BLK, zb, 0)
        for k0, sz in _ZCHUNKS:
            pltpu.sync_copy(rows_v.at[pl.ds(0, sz)], acc_sh.at[pl.ds(row0 + k0, sz)])
        plsc.subcore_barrier()

        def block(i, carry):
            base = tid * EP_TILE + i * BLK
            dbase = tid * (EP_TILE // CHUNK) + i * NCHUNK
            pltpu.sync_copy(src_hbm.at[pl.ds(base, BLK)], src_v)
            pltpu.sync_copy(dst_hbm.at[pl.ds(dbase, NCHUNK)], dst_v)
            offv = jnp.full((16,), off, jnp.int32)

            def addoff(k, cc):
                src_v[pl.ds(k * 16, 16)] = src_v[pl.ds(k * 16, 16)] + offv
                return cc
            lax.fori_loop(0, BLK // 16, addoff, 0)

            handles = []
            for j in range(NCHUNK):
                handles.append(pltpu.async_copy(
                    g_hbm.at[src_v.at[pl.ds(j * CHUNK, CHUNK)]],
                    rows_v.at[pl.ds(j * CHUNK, CHUNK)], gsem))
            for h in handles:
                h.wait()
            for j in range(NCHUNK):
                pltpu.sync_copy(rows_v.at[pl.ds(j * CHUNK, CHUNK)],
                                acc_sh.at[dst_v.at[j]], add=True)
            return carry
        lax.fori_loop(0, NB, block, 0)
        plsc.subcore_barrier()

        out_base = slice_id * NS + row0
        for k0, sz in _ZCHUNKS:
            pltpu.sync_copy(acc_sh.at[pl.ds(row0 + k0, sz)],
                            out_hbm.at[pl.ds(out_base + k0, sz)])
        if sl + 1 < spc:
            plsc.subcore_barrier()


def _sc_spmm(g_blocked, srcp, dst2d, n_slices):
    spc = n_slices // 2
    kern = pl.kernel(
        functools.partial(_spmm_body, spc),
        out_type=jax.ShapeDtypeStruct((n_slices * NS, 16), jnp.float32),
        mesh=plsc.VectorSubcoreMesh(core_axis_name="c", subcore_axis_name="s"),
        scratch_types=[
            pltpu.VMEM((BLK,), jnp.int32),
            pltpu.VMEM((NCHUNK, CHUNK), jnp.int32),
            pltpu.VMEM((BLK, 16), jnp.float32),
            pltpu.VMEM_SHARED((NS, 16), jnp.float32),
            pltpu.SemaphoreType.DMA,
        ],
        compiler_params=pltpu.CompilerParams(use_tc_tiling_on_sc=False),
    )
    return kern(g_blocked, srcp, dst2d)


def _block_table(g, n_slices):
    gb = g.reshape(N, n_slices, 16).transpose(1, 0, 2)
    gb = jnp.pad(gb, ((0, 0), (0, NP - N), (0, 0)))
    return gb.reshape(n_slices * NP, 16)


@jax.jit
def kernel(x, ei, targets, W1, b1, W2, b2, Wout, bout):
    src, dst = ei[0], ei[1]
    pad = E_PAD - E
    srcp = jnp.concatenate([src, jnp.full((pad,), N, jnp.int32)])
    dstp = jnp.concatenate([dst, jnp.zeros((pad,), jnp.int32)])
    dst2d = dstp.reshape(E_PAD // CHUNK, CHUNK)

    deg = jnp.zeros((N,), dtype=jnp.float32).at[dst].add(1.0) + 1.0
    dinv = jax.lax.rsqrt(deg)

    h0 = _matmul(x, W1)
    g1 = h0 * dinv[:, None]
    acc1 = _sc_spmm(_block_table(g1, 4), srcp, dst2d, 4)
    acc1 = acc1.reshape(4, NS, 16)[:, :N].transpose(1, 0, 2).reshape(N, 64)
    h = jax.nn.relu(dinv[:, None] * (acc1 + g1) + b1)

    h1 = _matmul(h, W2)
    g2 = h1 * dinv[:, None]
    acc2 = _sc_spmm(_block_table(g2, 2), srcp, dst2d, 2)
    acc2 = acc2.reshape(2, NS, 16)[:, :N].transpose(1, 0, 2).reshape(N, 32)
    z = dinv[:, None] * (acc2 + g2) + b2

    s = z[targets[0]]
    d = z[targets[1]]
    return (s * d) @ Wout + bout


# speedup vs baseline: 28.0993x; 2.0004x over previous
import functools
import jax
import jax.numpy as jnp
from jax import lax
from jax.experimental import pallas as pl
from jax.experimental.pallas import tpu as pltpu
from jax.experimental.pallas import tpu_sc as plsc

N = 100000
E = 3200000
BLK = 1024
CHUNK = 128
NCHUNK = BLK // CHUNK
E_PAD = 3211264
EP_TILE = E_PAD // 16
NB = EP_TILE // BLK
EP_DEG = E_PAD // 32
NB_DEG = EP_DEG // BLK
NS = 102400
RPT = NS // 16
_ZCHUNKS = tuple((k * BLK, BLK) for k in range(6)) + ((6 * BLK, 256),)

ROWS = 800
NT = N // ROWS
NST = NS // ROWS


def _deg_body(dst_hbm, out_hbm, ones_v, zero_v, dst_v, acc_sh):
    c = lax.axis_index("c")
    tid = lax.axis_index("s")
    row0 = tid * RPT

    def init(k, carry):
        ones_v[pl.ds(k * 16, 16)] = jnp.ones((16,), jnp.float32)
        zero_v[pl.ds(k * 16, 16)] = jnp.zeros((16,), jnp.float32)
        return carry
    lax.fori_loop(0, BLK // 16, init, 0)

    for k0, sz in _ZCHUNKS:
        pltpu.sync_copy(zero_v.at[pl.ds(0, sz)], acc_sh.at[pl.ds(row0 + k0, sz)])
    plsc.subcore_barrier()

    def block(i, carry):
        dbase = (c * 16 + tid) * (EP_DEG // CHUNK) + i * NCHUNK
        pltpu.sync_copy(dst_hbm.at[pl.ds(dbase, NCHUNK)], dst_v)
        for j in range(NCHUNK):
            pltpu.sync_copy(ones_v.at[pl.ds(j * CHUNK, CHUNK)],
                            acc_sh.at[dst_v.at[j]], add=True)
        return carry
    lax.fori_loop(0, NB_DEG, block, 0)
    plsc.subcore_barrier()

    for k0, sz in _ZCHUNKS:
        pltpu.sync_copy(acc_sh.at[pl.ds(row0 + k0, sz)],
                        out_hbm.at[pl.ds(c * NS + row0 + k0, sz)])


def _sc_degree(dst2d):
    kern = pl.kernel(
        _deg_body,
        out_type=jax.ShapeDtypeStruct((2 * NS,), jnp.float32),
        mesh=plsc.VectorSubcoreMesh(core_axis_name="c", subcore_axis_name="s"),
        scratch_types=[
            pltpu.VMEM((BLK,), jnp.float32),
            pltpu.VMEM((BLK,), jnp.float32),
            pltpu.VMEM((NCHUNK, CHUNK), jnp.int32),
            pltpu.VMEM_SHARED((NS,), jnp.float32),
        ],
        compiler_params=pltpu.CompilerParams(use_tc_tiling_on_sc=False),
    )
    return kern(dst2d)


def _spmm_body(spc, g_hbm, src_hbm, dst_hbm, out_hbm,
               src_v, dst_v, rows_v, acc_sh, gsem):
    c = lax.axis_index("c")
    tid = lax.axis_index("s")
    row0 = tid * RPT
    nsl = 2 * spc

    for sl in range(spc):
        slice_id = c * spc + sl

        def zb(k, carry):
            rows_v[k, :] = jnp.zeros((16,), jnp.float32)
            return carry
        lax.fori_loop(0, BLK, zb, 0)
        for k0, sz in _ZCHUNKS:
            pltpu.sync_copy(rows_v.at[pl.ds(0, sz)], acc_sh.at[pl.ds(row0 + k0, sz)])
        plsc.subcore_barrier()

        def block(i, carry):
            base = tid * EP_TILE + i * BLK
            dbase = tid * (EP_TILE // CHUNK) + i * NCHUNK
            pltpu.sync_copy(src_hbm.at[pl.ds(base, BLK)], src_v)
            pltpu.sync_copy(dst_hbm.at[pl.ds(dbase, NCHUNK)], dst_v)
            offv = jnp.full((16,), slice_id, jnp.int32)
            mulv = jnp.full((16,), nsl, jnp.int32)

            def addoff(k, cc):
                src_v[pl.ds(k * 16, 16)] = src_v[pl.ds(k * 16, 16)] * mulv + offv
                return cc
            lax.fori_loop(0, BLK // 16, addoff, 0)

            handles = []
            for j in range(NCHUNK):
                handles.append(pltpu.async_copy(
                    g_hbm.at[src_v.at[pl.ds(j * CHUNK, CHUNK)]],
                    rows_v.at[pl.ds(j * CHUNK, CHUNK)], gsem))
            for h in handles:
                h.wait()
            for j in range(NCHUNK):
                pltpu.sync_copy(rows_v.at[pl.ds(j * CHUNK, CHUNK)],
                                acc_sh.at[dst_v.at[j]], add=True)
            return carry
        lax.fori_loop(0, NB, block, 0)
        plsc.subcore_barrier()

        out_base = slice_id * NS + row0
        for k0, sz in _ZCHUNKS:
            pltpu.sync_copy(acc_sh.at[pl.ds(row0 + k0, sz)],
                            out_hbm.at[pl.ds(out_base + k0, sz)])
        if sl + 1 < spc:
            plsc.subcore_barrier()


def _sc_spmm(g_blocked, srcp, dst2d, n_slices):
    spc = n_slices // 2
    kern = pl.kernel(
        functools.partial(_spmm_body, spc),
        out_type=jax.ShapeDtypeStruct((n_slices * NS, 16), jnp.float32),
        mesh=plsc.VectorSubcoreMesh(core_axis_name="c", subcore_axis_name="s"),
        scratch_types=[
            pltpu.VMEM((BLK,), jnp.int32),
            pltpu.VMEM((NCHUNK, CHUNK), jnp.int32),
            pltpu.VMEM((BLK, 16), jnp.float32),
            pltpu.VMEM_SHARED((NS, 16), jnp.float32),
            pltpu.SemaphoreType.DMA,
        ],
        compiler_params=pltpu.CompilerParams(use_tc_tiling_on_sc=False),
    )
    return kern(g_blocked, srcp, dst2d)


def _tgt_body(z_hbm, t_hbm, out_hbm, idx_v, rows_v, sem):
    c = lax.axis_index("c")
    tid = lax.axis_index("s")
    base = (c * 16 + tid) * BLK
    pltpu.sync_copy(t_hbm.at[pl.ds(base, BLK)], idx_v)
    handles = []
    for j in range(NCHUNK):
        handles.append(pltpu.async_copy(
            z_hbm.at[idx_v.at[pl.ds(j * CHUNK, CHUNK)]],
            rows_v.at[pl.ds(j * CHUNK, CHUNK)], sem))
    for h in handles:
        h.wait()
    pltpu.sync_copy(rows_v, out_hbm.at[pl.ds(base, BLK)])


def _sc_gather_targets(z, tflat):
    kern = pl.kernel(
        _tgt_body,
        out_type=jax.ShapeDtypeStruct((32768, 32), jnp.float32),
        mesh=plsc.VectorSubcoreMesh(core_axis_name="c", subcore_axis_name="s"),
        scratch_types=[
            pltpu.VMEM((BLK,), jnp.int32),
            pltpu.VMEM((BLK, 32), jnp.float32),
            pltpu.SemaphoreType.DMA,
        ],
        compiler_params=pltpu.CompilerParams(use_tc_tiling_on_sc=False),
    )
    return kern(z, tflat)


def _k2_body(x_ref, w_ref, d0_ref, d1_ref, g_ref, dinv_ref):
    dinv = lax.rsqrt(d0_ref[...] + d1_ref[...] + 1.0)
    g_ref[...] = jnp.dot(x_ref[...], w_ref[...],
                         preferred_element_type=jnp.float32) * dinv
    dinv_ref[...] = dinv


def _tc_scale_l1(x, W1, degpart):
    return pl.pallas_call(
        _k2_body,
        grid=(NT,),
        in_specs=[
            pl.BlockSpec((ROWS, 18), lambda t: (t, 0)),
            pl.BlockSpec((18, 64), lambda t: (0, 0)),
            pl.BlockSpec((ROWS, 1), lambda t: (t, 0)),
            pl.BlockSpec((ROWS, 1), lambda t: (NST + t, 0)),
        ],
        out_specs=[
            pl.BlockSpec((ROWS, 64), lambda t: (t, 0)),
            pl.BlockSpec((ROWS, 1), lambda t: (t, 0)),
        ],
        out_shape=[
            jax.ShapeDtypeStruct((N, 64), jnp.float32),
            jax.ShapeDtypeStruct((N, 1), jnp.float32),
        ],
    )(x, W1, degpart, degpart)


def _k4_body(a0, a1, a2, a3, g_ref, dinv_ref, b1_ref, w_ref, out_ref):
    dinv = dinv_ref[...]
    acc = jnp.concatenate([a[...] for a in (a0, a1, a2, a3)], axis=1)
    h = jax.nn.relu(dinv * (acc + g_ref[...]) + b1_ref[...])
    out_ref[...] = jnp.dot(h, w_ref[...], preferred_element_type=jnp.float32) * dinv


def _tc_layer2_tables(acc1, g1flat, dinv, b1, W2):
    in_acc = [pl.BlockSpec((ROWS, 16), functools.partial(
        lambda s, t: (s * NST + t, 0), s)) for s in range(4)]
    return pl.pallas_call(
        _k4_body,
        grid=(NT,),
        in_specs=in_acc + [
            pl.BlockSpec((ROWS, 64), lambda t: (t, 0)),
            pl.BlockSpec((ROWS, 1), lambda t: (t, 0)),
            pl.BlockSpec((1, 64), lambda t: (0, 0)),
            pl.BlockSpec((64, 32), lambda t: (0, 0)),
        ],
        out_specs=pl.BlockSpec((ROWS, 32), lambda t: (t, 0)),
        out_shape=jax.ShapeDtypeStruct((N, 32), jnp.float32),
    )(acc1, acc1, acc1, acc1, g1flat, dinv, b1.reshape(1, 64), W2)


def _k6_body(a0, a1, g_ref, dinv_ref, b2_ref, z_ref):
    dinv = dinv_ref[...]
    acc = jnp.concatenate([a[...] for a in (a0, a1)], axis=1)
    z_ref[...] = dinv * (acc + g_ref[...]) + b2_ref[...]


def _tc_assemble_z(acc2, g2flat, dinv, b2):
    in_acc = [pl.BlockSpec((ROWS, 16), functools.partial(
        lambda s, t: (s * NST + t, 0), s)) for s in range(2)]
    return pl.pallas_call(
        _k6_body,
        grid=(NT,),
        in_specs=in_acc + [
            pl.BlockSpec((ROWS, 32), lambda t: (t, 0)),
            pl.BlockSpec((ROWS, 1), lambda t: (t, 0)),
            pl.BlockSpec((1, 32), lambda t: (0, 0)),
        ],
        out_specs=pl.BlockSpec((ROWS, 32), lambda t: (t, 0)),
        out_shape=jax.ShapeDtypeStruct((N, 32), jnp.float32),
    )(acc2, acc2, g2flat, dinv, b2.reshape(1, 32))


def _k8_body(s_ref, d_ref, w_ref, bout_ref, o_ref):
    prod = s_ref[...] * d_ref[...] * w_ref[...]
    o_ref[...] = jnp.sum(prod, axis=1, keepdims=True) + bout_ref[...]


def _tc_score(pairs, Wout, bout):
    return pl.pallas_call(
        _k8_body,
        grid=(8,),
        in_specs=[
            pl.BlockSpec((2048, 32), lambda t: (t, 0)),
            pl.BlockSpec((2048, 32), lambda t: (8 + t, 0)),
            pl.BlockSpec((1, 32), lambda t: (0, 0)),
            pl.BlockSpec((1, 1), lambda t: (0, 0)),
        ],
        out_specs=pl.BlockSpec((2048, 1), lambda t: (t, 0)),
        out_shape=jax.ShapeDtypeStruct((16384, 1), jnp.float32),
    )(pairs, pairs, Wout.reshape(1, 32), bout.reshape(1, 1))


@jax.jit
def kernel(x, ei, targets, W1, b1, W2, b2, Wout, bout):
    src, dst = ei[0], ei[1]
    pad = E_PAD - E
    srcp = jnp.concatenate([src, jnp.zeros((pad,), jnp.int32)])
    dstp = jnp.concatenate([dst, jnp.full((pad,), N, jnp.int32)])
    dst2d = dstp.reshape(E_PAD // CHUNK, CHUNK)

    degpart = _sc_degree(dst2d).reshape(2 * NS, 1)

    g1flat, dinv = _tc_scale_l1(x, W1, degpart)
    acc1 = _sc_spmm(g1flat.reshape(4 * N, 16), srcp, dst2d, 4)
    g2flat = _tc_layer2_tables(acc1, g1flat, dinv, b1, W2)
    acc2 = _sc_spmm(g2flat.reshape(2 * N, 16), srcp, dst2d, 2)
    z = _tc_assemble_z(acc2, g2flat, dinv, b2)
    pairs = _sc_gather_targets(z, targets.reshape(-1))
    return _tc_score(pairs, Wout, bout)
